# baseline (speedup 1.0000x reference)
import jax
import jax.numpy as jnp
from jax.experimental import pallas as pl
from jax.experimental.pallas import tpu as pltpu

B, C = 1024, 7
CC = C * C


BI = 512


def _body(w_ref, xT_ref, xrep_ref, out_ref, oh_ref):
    c = pl.program_id(0)
    k = pl.program_id(1)
    ib = pl.program_id(2)

    @pl.when((c == 0) & (k == 0) & (ib == 0))
    def _():
        xi = xrep_ref[...]
        idx = (xi * (xi >= 0).astype(xi.dtype)).astype(jnp.int32)
        lio = jax.lax.broadcasted_iota(jnp.int32, (B, CC), 1)
        oh_ref[...] = (idx == lio % C).astype(jnp.bfloat16)

    ck = c * C + k
    sel = (jax.lax.broadcasted_iota(jnp.int32, (CC, B), 0) == ck
           ).astype(jnp.bfloat16)
    m = jax.lax.dot_general(
        oh_ref[pl.ds(ib * BI, BI), :], sel,
        dimension_numbers=(((1,), (0,)), ((), ())),
        preferred_element_type=jnp.float32,
    )
    xk = xT_ref[...].reshape(1, B)
    wk = w_ref[k]
    out_ref[...] = (jnp.abs(xk - m) * wk).reshape(1, 1, BI, B)


def kernel(input, target, code_weights):
    x = input.reshape(B, C)
    xT = x.T.reshape(C, 1, B)
    xrep = jnp.repeat(x, C, axis=1)

    out = pl.pallas_call(
        _body,
        grid=(C, C, B // BI),
        in_specs=[
            pl.BlockSpec(memory_space=pltpu.SMEM),
            pl.BlockSpec((1, 1, B), lambda c, k, ib: (k, 0, 0)),
            pl.BlockSpec((B, CC), lambda c, k, ib: (0, 0)),
        ],
        out_specs=pl.BlockSpec((1, 1, BI, B), lambda c, k, ib: (c, k, ib, 0)),
        out_shape=jax.ShapeDtypeStruct((C, C, B, B), jnp.float32),
        scratch_shapes=[pltpu.VMEM((B, CC), jnp.bfloat16)],
    )(code_weights, xT, xrep)
    return out.transpose(2, 3, 0, 1)


# speedup vs baseline: 1.3173x; 1.3173x over previous
import jax
import jax.numpy as jnp
from jax.experimental import pallas as pl
from jax.experimental.pallas import tpu as pltpu

B, C = 1024, 7
CC = C * C


BI = 1024


def _body(w_ref, xT_ref, xrep_ref, out_ref, oh_ref):
    c = pl.program_id(0)
    k = pl.program_id(1)
    ib = pl.program_id(2)

    @pl.when((c == 0) & (k == 0) & (ib == 0))
    def _():
        xi = xrep_ref[...]
        idx = (xi * (xi >= 0).astype(xi.dtype)).astype(jnp.int32)
        lio = jax.lax.broadcasted_iota(jnp.int32, (B, CC), 1)
        oh_ref[...] = (idx == lio % C).astype(jnp.bfloat16)

    ck = c * C + k
    sel = (jax.lax.broadcasted_iota(jnp.int32, (CC, B), 0) == ck
           ).astype(jnp.bfloat16)
    m = jax.lax.dot_general(
        oh_ref[pl.ds(ib * BI, BI), :], sel,
        dimension_numbers=(((1,), (0,)), ((), ())),
        preferred_element_type=jnp.float32,
    )
    xk = xT_ref[...].reshape(1, B)
    wk = w_ref[k]
    out_ref[...] = (jnp.abs(xk - m) * wk).reshape(1, 1, BI, B)


def kernel(input, target, code_weights):
    x = input.reshape(B, C)
    xT = x.T.reshape(C, 1, B)
    xrep = jnp.repeat(x, C, axis=1)

    out = pl.pallas_call(
        _body,
        grid=(C, C, B // BI),
        in_specs=[
            pl.BlockSpec(memory_space=pltpu.SMEM),
            pl.BlockSpec((1, 1, B), lambda c, k, ib: (k, 0, 0)),
            pl.BlockSpec((B, CC), lambda c, k, ib: (0, 0)),
        ],
        out_specs=pl.BlockSpec((1, 1, BI, B), lambda c, k, ib: (c, k, ib, 0)),
        out_shape=jax.ShapeDtypeStruct((C, C, B, B), jnp.float32),
        scratch_shapes=[pltpu.VMEM((B, CC), jnp.bfloat16)],
    )(code_weights, xT, xrep)
    return out.transpose(2, 3, 0, 1)
